# baseline (speedup 1.0000x reference)
import functools

import jax
import jax.numpy as jnp
from jax import lax
from jax.experimental import pallas as pl
from jax.experimental.pallas import tpu as pltpu
from jax.experimental.pallas import tpu_sc as plsc

CHUNK = 128
NBUF = 2
NC, NS, L = 2, 16, 16
NW = NC * NS

_mesh = plsc.VectorSubcoreMesh(core_axis_name="c", subcore_axis_name="s")


def _make_gather(n: int, d: int):
    rows_per_w = n // NW
    assert rows_per_w % (CHUNK * NBUF) == 0
    n_chunks = rows_per_w // CHUNK

    @functools.partial(
        pl.kernel,
        mesh=_mesh,
        compiler_params=pltpu.CompilerParams(needs_layout_passes=False),
        out_type=jax.ShapeDtypeStruct((n, d), jnp.float32),
        scratch_types=[
            pltpu.VMEM((rows_per_w,), jnp.int32),
            pltpu.VMEM((NBUF, CHUNK), jnp.int32),
            pltpu.VMEM((NBUF, CHUNK, 2 * d), jnp.float32),
            pltpu.VMEM((NBUF, CHUNK, d), jnp.float32),
            pltpu.SemaphoreType.DMA,
            pltpu.SemaphoreType.DMA,
        ],
    )
    def _gather(idx_hbm, tp_hbm, out_hbm, idx_v, pair_v, pairs_v, out_v,
                sem0, sem1):
        wid = lax.axis_index("s") * NC + lax.axis_index("c")
        base = wid * rows_per_w
        sems = [sem0, sem1]

        pltpu.sync_copy(idx_hbm.at[pl.ds(base, rows_per_w)], idx_v)
        iota = lax.iota(jnp.int32, L)

        def fire(c, buf):
            for g in range(CHUNK // L):
                iv = idx_v[pl.ds(c * CHUNK + g * L, L)]
                pair_v[buf, pl.ds(g * L, L)] = lax.shift_right_logical(iv, 1)
            pltpu.async_copy(
                tp_hbm.at[pair_v.at[buf]], pairs_v.at[buf], sems[buf]
            )

        def drain(buf):
            pltpu.make_async_copy(
                tp_hbm.at[pl.ds(0, CHUNK)], pairs_v.at[buf], sems[buf]
            ).wait()

        def compact(c, buf):
            for g in range(CHUNK // L):
                iv = idx_v[pl.ds(c * CHUNK + g * L, L)]
                col0 = lax.bitwise_and(iv, 1) * d
                rowvec = g * L + iota

                def body(j, carry):
                    v = plsc.load_gather(pairs_v.at[buf], [rowvec, col0 + j])
                    plsc.store_scatter(
                        out_v.at[buf],
                        [rowvec, jnp.full((L,), 0, jnp.int32) + j], v,
                    )
                    return carry

                lax.fori_loop(0, d, body, 0, unroll=8)

        fire(0, 0)

        def pair_step(p, carry):
            c0 = p * NBUF
            for b in range(NBUF):
                c = c0 + b

                @pl.when(c + 1 < n_chunks)
                def _():
                    fire(c + 1, (b + 1) % NBUF)

                drain(b)
                compact(c, b)
                pltpu.sync_copy(
                    out_v.at[b], out_hbm.at[pl.ds(base + c * CHUNK, CHUNK)]
                )
            return carry

        lax.fori_loop(0, n_chunks // NBUF, pair_step, 0)

    return _gather


def kernel(x, table):
    b, h = x.shape
    v, d = table.shape
    flat = x.reshape(-1).astype(jnp.int32)
    tp = table.reshape(v // 2, 2 * d)
    out = _make_gather(flat.shape[0], d)(flat, tp)
    return out.reshape(b, h, d)


# speedup vs baseline: 2.4208x; 2.4208x over previous
import functools

import jax
import jax.numpy as jnp
from jax import lax
from jax.experimental import pallas as pl
from jax.experimental.pallas import tpu as pltpu
from jax.experimental.pallas import tpu_sc as plsc

CHUNK = 128
NBUF = 2
NC, NS, L = 2, 16, 16
NW = NC * NS

_mesh = plsc.VectorSubcoreMesh(core_axis_name="c", subcore_axis_name="s")


def _make_gather(n: int, d: int):
    rows_per_w = n // NW
    assert rows_per_w % (CHUNK * NBUF) == 0
    n_chunks = rows_per_w // CHUNK

    @functools.partial(
        pl.kernel,
        mesh=_mesh,
        compiler_params=pltpu.CompilerParams(needs_layout_passes=False),
        out_type=jax.ShapeDtypeStruct((n, d), jnp.float32),
        scratch_types=[
            pltpu.VMEM((rows_per_w,), jnp.int32),
            pltpu.VMEM((NBUF, CHUNK), jnp.int32),
            pltpu.VMEM((NBUF, CHUNK, 2 * d), jnp.float32),
            pltpu.VMEM((NBUF, CHUNK, d), jnp.float32),
            pltpu.SemaphoreType.DMA,
            pltpu.SemaphoreType.DMA,
        ],
    )
    def _gather(idx_hbm, tp_hbm, out_hbm, idx_v, pair_v, pairs_v, out_v,
                sem0, sem1):
        wid = lax.axis_index("s") * NC + lax.axis_index("c")
        base = wid * rows_per_w
        sems = [sem0, sem1]

        pltpu.sync_copy(idx_hbm.at[pl.ds(base, rows_per_w)], idx_v)

        def fire(c, buf):
            for g in range(CHUNK // L):
                iv = idx_v[pl.ds(c * CHUNK + g * L, L)]
                pair_v[buf, pl.ds(g * L, L)] = lax.shift_right_logical(iv, 1)
            pltpu.async_copy(
                tp_hbm.at[pair_v.at[buf]], pairs_v.at[buf], sems[buf]
            )

        def drain(buf):
            pltpu.make_async_copy(
                tp_hbm.at[pl.ds(0, CHUNK)], pairs_v.at[buf], sems[buf]
            ).wait()

        def compact(c, buf):
            def group(g, carry):
                for u in range(L):
                    r = g * L + u
                    hv = plsc.load_gather(
                        idx_v, [jnp.full((L,), c * CHUNK + r, jnp.int32)]
                    )
                    m = lax.bitwise_and(hv, 1) != 0
                    for j4 in range(d // L):
                        lo = pairs_v[buf, r, pl.ds(j4 * L, L)]
                        hi = pairs_v[buf, r, pl.ds(d + j4 * L, L)]
                        out_v[buf, r, pl.ds(j4 * L, L)] = jnp.where(m, hi, lo)
                return carry

            lax.fori_loop(0, CHUNK // L, group, 0)

        fire(0, 0)

        def pair_step(p, carry):
            c0 = p * NBUF
            for b in range(NBUF):
                c = c0 + b

                @pl.when(c + 1 < n_chunks)
                def _():
                    fire(c + 1, (b + 1) % NBUF)

                drain(b)
                compact(c, b)
                pltpu.sync_copy(
                    out_v.at[b], out_hbm.at[pl.ds(base + c * CHUNK, CHUNK)]
                )
            return carry

        lax.fori_loop(0, n_chunks // NBUF, pair_step, 0)

    return _gather


def kernel(x, table):
    b, h = x.shape
    v, d = table.shape
    flat = x.reshape(-1).astype(jnp.int32)
    tp = table.reshape(v // 2, 2 * d)
    out = _make_gather(flat.shape[0], d)(flat, tp)
    return out.reshape(b, h, d)
